# Initial kernel scaffold; baseline (speedup 1.0000x reference)
import jax
import jax.numpy as jnp
from jax.experimental import pallas as pl


def kernel(input_ids, segment_ids, token_emb, pos_emb, seg_emb, ln_gamma, ln_beta):
    raise NotImplementedError("write your pallas kernel here")


# speedup vs baseline: 2.8198x; 2.8198x over previous
import functools

import jax
import jax.numpy as jnp
from jax import lax
from jax.experimental import pallas as pl
from jax.experimental.pallas import tpu as pltpu
from jax.experimental.pallas import tpu_sc as plsc

_VOCAB = 100000
_HIDDEN = 128
_SEQ = 200
_SEQ_PAD = 208
_BATCH = 1024
_EPS = 1e-12
_NC = 2
_NS = 16
_NW = _NC * _NS
_SEQS_PER_W = _BATCH // _NW
_NJ = _HIDDEN // 16


def _rsqrt_newton(v):
    i = plsc.bitcast(v, jnp.int32)
    i = jnp.int32(0x5F3759DF) - lax.shift_right_logical(i, 1)
    y = plsc.bitcast(i, jnp.float32)
    xh = v * jnp.float32(0.5)
    for _ in range(3):
        y = y * (jnp.float32(1.5) - xh * y * y)
    return y


def _sc_embed_ln(token_emb, possego, ids_pad, comb_pad, gamma, beta):
    mesh = plsc.VectorSubcoreMesh(core_axis_name="c", subcore_axis_name="s")

    @functools.partial(
        pl.kernel,
        mesh=mesh,
        compiler_params=pltpu.CompilerParams(needs_layout_passes=False),
        out_type=jax.ShapeDtypeStruct((_BATCH, _SEQ, _HIDDEN), jnp.float32),
        scratch_types=[
            pltpu.VMEM((_SEQ_PAD, _HIDDEN), jnp.float32),
            pltpu.VMEM((400, _HIDDEN), jnp.float32),
            pltpu.VMEM((2, _SEQ_PAD // 2), jnp.int32),
            pltpu.VMEM((_SEQ_PAD,), jnp.int32),
            pltpu.VMEM((_HIDDEN,), jnp.float32),
            pltpu.VMEM((_HIDDEN,), jnp.float32),
            pltpu.SemaphoreType.DMA,
            pltpu.SemaphoreType.DMA,
        ],
    )
    def k(tok_hbm, pose_hbm, ids_hbm, comb_hbm, gam_hbm, bet_hbm, out_hbm,
          buf, pose_v, ids_v, comb_v, gam_v, bet_v, sem0, sem1):
        wid = lax.axis_index("s") * _NC + lax.axis_index("c")
        pltpu.sync_copy(pose_hbm, pose_v)
        pltpu.sync_copy(gam_hbm, gam_v)
        pltpu.sync_copy(bet_hbm, bet_v)
        g_regs = [gam_v[pl.ds(16 * j, 16)] for j in range(_NJ)]
        b_regs = [bet_v[pl.ds(16 * j, 16)] for j in range(_NJ)]
        lanes = lax.iota(jnp.int32, 16)
        inv_h = jnp.float32(1.0 / _HIDDEN)
        eps = jnp.float32(_EPS)

        def seq_body(s, carry):
            b = wid * _SEQS_PER_W + s
            pltpu.sync_copy(ids_hbm.at[b], ids_v)
            pltpu.sync_copy(comb_hbm.at[b], comb_v)
            cp0 = pltpu.async_copy(tok_hbm.at[ids_v.at[0]],
                                   buf.at[pl.ds(0, _SEQ_PAD // 2)], sem0)
            cp1 = pltpu.async_copy(tok_hbm.at[ids_v.at[1]],
                                   buf.at[pl.ds(_SEQ_PAD // 2, _SEQ_PAD // 2)], sem1)
            cp0.wait()
            cp1.wait()

            def grp_body(g, carry2):
                base = g * 16
                comb_vec = comb_v[pl.ds(base, 16)]
                for i in range(16):
                    tok = base + i
                    off = jnp.sum(jnp.where(lanes == i, comb_vec, 0))
                    x = []
                    for j in range(_NJ):
                        x.append(buf[tok, pl.ds(16 * j, 16)]
                                 + pose_v[off, pl.ds(16 * j, 16)])
                    ssum = x[0]
                    for j in range(1, _NJ):
                        ssum = ssum + x[j]
                    qsum = x[0] * x[0]
                    for j in range(1, _NJ):
                        qsum = qsum + x[j] * x[j]
                    s_tot = jnp.sum(ssum)
                    q_tot = jnp.sum(qsum)
                    meanv = jnp.full((16,), s_tot, jnp.float32) * inv_h
                    qv = jnp.full((16,), q_tot, jnp.float32) * inv_h
                    varv = qv - meanv * meanv
                    rstd = _rsqrt_newton(varv + eps)
                    for j in range(_NJ):
                        buf[tok, pl.ds(16 * j, 16)] = (
                            (x[j] - meanv) * (rstd * g_regs[j]) + b_regs[j])
                return carry2

            lax.fori_loop(0, _SEQ_PAD // 16, grp_body, 0)
            pltpu.sync_copy(buf.at[pl.ds(0, _SEQ)], out_hbm.at[b])
            return carry

        lax.fori_loop(0, _SEQS_PER_W, seq_body, 0)

    return k(token_emb, possego, ids_pad, comb_pad, gamma, beta)


def kernel(input_ids, segment_ids, token_emb, pos_emb, seg_emb, ln_gamma, ln_beta):
    input_ids = input_ids.astype(jnp.int32)
    segment_ids = segment_ids.astype(jnp.int32)
    possego = (pos_emb[:_SEQ, None, :] + seg_emb[None, :, :]).reshape(2 * _SEQ, _HIDDEN)
    pad = _SEQ_PAD - _SEQ
    ids_pad = jnp.pad(input_ids, ((0, 0), (0, pad))).reshape(_BATCH, 2, _SEQ_PAD // 2)
    comb_pad = jnp.pad(2 * jnp.arange(_SEQ, dtype=jnp.int32)[None, :] + segment_ids,
                       ((0, 0), (0, pad)))
    return _sc_embed_ln(token_emb, possego, ids_pad, comb_pad, ln_gamma, ln_beta)
